# baseline (speedup 1.0000x reference)
import functools

import jax
import jax.numpy as jnp
from jax import lax
from jax.experimental import pallas as pl
from jax.experimental.pallas import tpu as pltpu
from jax.experimental.pallas import tpu_sc as plsc

B = 4
S = 8192
H = 1024
LANES = 16
NC = 2
NS = 16
NW = NC * NS
TOK = B * S
TPW = TOK // NW
K = 8
NCHUNK = TPW // K
NBUF = 4
NITER = NCHUNK // NBUF
VECS_PER_ROW = H // LANES


def _add_rows_inplace(dst, src):
    def row_body(r, carry):
        for j in range(VECS_PER_ROW):
            col = j * LANES
            dst[r, pl.ds(col, LANES)] = (
                dst[r, pl.ds(col, LANES)] + src[r, pl.ds(col, LANES)]
            )
        return carry
    lax.fori_loop(0, K, row_body, 0)


def _sc_kernel(ids_hbm, t1_hbm, t2_hbm, out_hbm,
               idx1, idx2, *bufs_and_sems):
    a = bufs_and_sems[0:NBUF]
    bsuf = bufs_and_sems[NBUF:2 * NBUF]
    gsem = bufs_and_sems[2 * NBUF:3 * NBUF]
    ssem = bufs_and_sems[3 * NBUF:4 * NBUF]

    wid = lax.axis_index("s") * NC + lax.axis_index("c")
    b = wid // (S // TPW)
    sbase = (wid % (S // TPW)) * TPW
    pltpu.sync_copy(ids_hbm.at[pl.ds(b * 2 * S + sbase, TPW)], idx1)
    pltpu.sync_copy(ids_hbm.at[pl.ds(b * 2 * S + S + sbase, TPW)], idx2)

    def gathers(c, t):
        pltpu.async_copy(t1_hbm.at[idx1.at[pl.ds(c * K, K)]], a[t], gsem[t])
        pltpu.async_copy(t2_hbm.at[idx2.at[pl.ds(c * K, K)]], bsuf[t], gsem[t])

    def wait_gathers(t):
        pltpu.make_async_copy(t1_hbm.at[pl.ds(0, K)], a[t], gsem[t]).wait()
        pltpu.make_async_copy(t2_hbm.at[pl.ds(0, K)], bsuf[t], gsem[t]).wait()

    def wait_store(t):
        pltpu.make_async_copy(out_hbm.at[pl.ds(0, K)], a[t], ssem[t]).wait()

    for t in range(3):
        gathers(t, t)

    def body(i, carry):
        for t in range(NBUF):
            c = NBUF * i + t
            nt = (t + 3) % NBUF

            @pl.when(c >= 1)
            def _():
                wait_store(nt)

            @pl.when(c < NCHUNK - 3)
            def _():
                gathers(c + 3, nt)

            wait_gathers(t)
            _add_rows_inplace(a[t], bsuf[t])
            pltpu.async_copy(a[t], out_hbm.at[pl.ds(wid * TPW + c * K, K)],
                             ssem[t])
        return carry

    lax.fori_loop(0, NITER, body, 0)
    wait_store((NCHUNK - 1) % NBUF)


@jax.jit
def _run(ids_flat, position_table, block_position_table):
    mesh = plsc.VectorSubcoreMesh(core_axis_name="c", subcore_axis_name="s")
    fn = functools.partial(
        pl.kernel,
        mesh=mesh,
        out_type=jax.ShapeDtypeStruct((TOK, H), jnp.float32),
        scratch_types=(
            [pltpu.VMEM((TPW,), jnp.int32)] * 2
            + [pltpu.VMEM((K, H), jnp.float32)] * (2 * NBUF)
            + [pltpu.SemaphoreType.DMA] * (2 * NBUF)
        ),
    )(_sc_kernel)
    return fn(ids_flat, position_table, block_position_table)


def kernel(position_ids, position_table, block_position_table):
    ids_flat = position_ids.astype(jnp.int32).reshape(-1)
    out = _run(ids_flat, position_table, block_position_table)
    return out.reshape(B, S, H)
